# baseline (speedup 1.0000x reference)
import functools

import jax
import jax.numpy as jnp
import numpy as np
from jax import lax
from jax.experimental import pallas as pl
from jax.experimental.pallas import tpu as pltpu
from jax.experimental.pallas import tpu_sc as plsc

N_NODES = 10000
D = 128
B = 10000
BP = 10240
E = 160000
EP = 163840
GW = 64
NG = EP // (32 * GW)
G0 = 104
G1 = 160 - G0
ROWS_PER_TILE = BP // 16
NODE_ROWS = BP // GW

_SIGMA = np.empty((D,), np.int64)
for _k in range(D // 32):
  for _i in range(16):
    _SIGMA[32 * _k + _i] = 32 * _k + 2 * _i
    _SIGMA[32 * _k + 16 + _i] = 32 * _k + 2 * _i + 1


def _sc_aggregate(nodes2d, dst0, src0, dst1, src1, feat_table, tbl32):
  mesh = plsc.VectorSubcoreMesh(
      core_axis_name="c", subcore_axis_name="s", num_cores=2, num_subcores=16)

  @functools.partial(
      pl.kernel,
      out_type=[
          jax.ShapeDtypeStruct((BP, D), jnp.float32),
          jax.ShapeDtypeStruct((2, 2, BP, D), jnp.float32),
          jax.ShapeDtypeStruct((2, 2, BP), jnp.float32),
      ],
      mesh=mesh,
      compiler_params=pltpu.CompilerParams(use_tc_tiling_on_sc=False),
      scratch_types=[
          pltpu.VMEM((max(G0, G1), GW), jnp.int32),
          pltpu.VMEM((max(G0, G1), GW), jnp.int32),
          pltpu.VMEM((NODE_ROWS // 32, GW), jnp.int32),
          pltpu.VMEM((2, GW, D // 2), jnp.int32),
          pltpu.VMEM((2, GW, D), jnp.float32),
          pltpu.VMEM((GW,), jnp.float32),
          pltpu.VMEM((16, D), jnp.float32),
          pltpu.VMEM((ROWS_PER_TILE,), jnp.float32),
          pltpu.VMEM_SHARED((BP, D), jnp.float32),
          pltpu.VMEM_SHARED((BP,), jnp.float32),
          pltpu.SemaphoreType.DMA,
          pltpu.SemaphoreType.DMA((2,)),
          pltpu.SemaphoreType.DMA((2,)),
      ],
  )
  def body(nodes_hbm, d0_hbm, s0_hbm, d1_hbm, s1_hbm, table_hbm, tbl32_hbm,
           self_hbm, sums_hbm, cnts_hbm,
           dstbuf, srcbuf, nodebuf, brows, frows, ones, zrow, zcnt,
           acc, cntacc, sem, gsem, ssem):
    core = lax.axis_index("c")
    sid = lax.axis_index("s")
    wid = sid * 2 + core

    for i in range(GW // 16):
      ones[pl.ds(i * 16, 16)] = jnp.ones((16,), jnp.float32)
    for r in range(16):
      for i in range(8):
        zrow[r, pl.ds(i * 16, 16)] = jnp.zeros((16,), jnp.float32)
    for i in range(ROWS_PER_TILE // 16):
      zcnt[pl.ds(i * 16, 16)] = jnp.zeros((16,), jnp.float32)

    NT = NODE_ROWS // 32
    pltpu.sync_copy(nodes_hbm.at[pl.ds(wid * NT, NT)], nodebuf)
    pltpu.async_copy(table_hbm.at[nodebuf.at[0]], frows.at[0], gsem.at[0])
    pltpu.async_copy(table_hbm.at[nodebuf.at[1]], frows.at[1], gsem.at[1])
    for t in range(NT):
      b = t % 2
      pltpu.make_async_copy(
          table_hbm.at[nodebuf.at[t]], frows.at[b], gsem.at[b]).wait()
      pltpu.sync_copy(
          frows.at[b], self_hbm.at[pl.ds((wid * NT + t) * GW, GW)])
      if t + 2 < NT:
        pltpu.async_copy(
            table_hbm.at[nodebuf.at[t + 2]], frows.at[b], gsem.at[b])

    for rel, (d_hbm, s_hbm) in enumerate(((d0_hbm, s0_hbm), (d1_hbm, s1_hbm))):
      gstart = lax.select(core == 0, sid * G0, 16 * G0 + sid * G1)
      ng = lax.select(core == 0, G0, G1)
      idx_dma_d = pltpu.async_copy(
          d_hbm.at[pl.ds(gstart, max(G0, G1))], dstbuf, sem)
      idx_dma_s = pltpu.async_copy(
          s_hbm.at[pl.ds(gstart, max(G0, G1))], srcbuf, sem)

      def zero_step(j, zc):
        pltpu.sync_copy(zrow, acc.at[pl.ds(sid * ROWS_PER_TILE + j * 16, 16)])
        return zc
      lax.fori_loop(0, ROWS_PER_TILE // 16, zero_step, 0)
      pltpu.sync_copy(zcnt, cntacc.at[pl.ds(sid * ROWS_PER_TILE, ROWS_PER_TILE)])
      idx_dma_d.wait()
      idx_dma_s.wait()
      plsc.subcore_barrier()

      pltpu.async_copy(tbl32_hbm.at[srcbuf.at[0]], brows.at[0], gsem.at[0])
      pltpu.async_copy(tbl32_hbm.at[srcbuf.at[1]], brows.at[1], gsem.at[1])

      def edge_step(j, carry):
        buf = lax.rem(j, 2)
        pltpu.make_async_copy(
            tbl32_hbm.at[srcbuf.at[j]], brows.at[buf], gsem.at[buf]).wait()
        @pl.when(j >= 2)
        def _drain():
          pltpu.make_async_copy(
              frows.at[buf], acc.at[dstbuf.at[j]], ssem.at[buf]).wait()
        for r in range(GW):
          for c in range(D // 32):
            u = brows[buf, r, pl.ds(16 * c, 16)]
            lo = lax.bitcast_convert_type(u << 16, jnp.float32)
            hi = lax.bitcast_convert_type(u & (-65536), jnp.float32)
            frows[buf, r, pl.ds(32 * c, 16)] = lo
            frows[buf, r, pl.ds(32 * c + 16, 16)] = hi
        @pl.when(j + 2 < ng)
        def _prefetch():
          pltpu.async_copy(
              tbl32_hbm.at[srcbuf.at[j + 2]], brows.at[buf], gsem.at[buf])
        pltpu.async_copy(frows.at[buf], acc.at[dstbuf.at[j]], ssem.at[buf],
                         add=True)
        pltpu.sync_copy(ones, cntacc.at[dstbuf.at[j]], add=True)
        return carry
      lax.fori_loop(0, ng, edge_step, 0)
      for b in range(2):
        pltpu.make_async_copy(
            frows.at[b], acc.at[dstbuf.at[b]], ssem.at[b]).wait()
      plsc.subcore_barrier()

      sl = pl.ds(sid * ROWS_PER_TILE, ROWS_PER_TILE)
      pltpu.sync_copy(acc.at[sl], sums_hbm.at[rel, core].at[sl])
      pltpu.sync_copy(cntacc.at[sl], cnts_hbm.at[rel, core].at[sl])

  return body(nodes2d, dst0, src0, dst1, src1, feat_table, tbl32)


def _tc_mlp_body(self_ref, sums_ref, cnts_ref, w1t_ref, b1_ref, w2t_ref,
                 b2_ref, out_ref):
  s0 = sums_ref[0, 0] + sums_ref[0, 1]
  c0 = cnts_ref[0, 0] + cnts_ref[0, 1]
  n0 = s0 / jnp.clip(c0, 1.0, None)
  s1 = sums_ref[1, 0] + sums_ref[1, 1]
  c1 = cnts_ref[1, 0] + cnts_ref[1, 1]
  n1 = s1 / jnp.clip(c1, 1.0, None)
  h = jnp.dot(self_ref[...], w1t_ref[pl.ds(0, D)],
              preferred_element_type=jnp.float32)
  h += jnp.dot(n0, w1t_ref[pl.ds(D, D)], preferred_element_type=jnp.float32)
  h += jnp.dot(n1, w1t_ref[pl.ds(2 * D, D)], preferred_element_type=jnp.float32)
  h = jnp.tanh(h + b1_ref[...])
  out_ref[...] = jnp.dot(h, w2t_ref[...],
                         preferred_element_type=jnp.float32) + b2_ref[...]


def _tc_mlp(self_feats, sums, cnts4, w1t, b1, w2t, b2):
  R = 2000
  return pl.pallas_call(
      _tc_mlp_body,
      grid=(B // R,),
      in_specs=[
          pl.BlockSpec((R, D), lambda i: (i, 0)),
          pl.BlockSpec((2, 2, R, D), lambda i: (0, 0, i, 0)),
          pl.BlockSpec((2, 2, R, 1), lambda i: (0, 0, i, 0)),
          pl.BlockSpec((3 * D, D), lambda i: (0, 0)),
          pl.BlockSpec((1, D), lambda i: (0, 0)),
          pl.BlockSpec((D, D), lambda i: (0, 0)),
          pl.BlockSpec((1, D), lambda i: (0, 0)),
      ],
      out_specs=pl.BlockSpec((R, D), lambda i: (i, 0)),
      out_shape=jax.ShapeDtypeStruct((B, D), jnp.float32),
  )(self_feats, sums, cnts4, w1t, b1, w2t, b2)


def kernel(nodes, edge_index_0, edge_index_1, feat_table, W1, b1, W2, b2):
  i32 = jnp.int32
  nodes_p = jnp.concatenate(
      [nodes.astype(i32), jnp.zeros((BP - B,), i32)]).reshape(NODE_ROWS, GW)

  def prep_edges(ei):
    dst = jnp.concatenate([ei[0].astype(i32), jnp.full((EP - E,), B, i32)])
    src = jnp.concatenate([ei[1].astype(i32), jnp.zeros((EP - E,), i32)])
    return dst.reshape(EP // GW, GW), src.reshape(EP // GW, GW)

  d0, s0 = prep_edges(edge_index_0)
  d1, s1 = prep_edges(edge_index_1)
  tbl16 = feat_table.astype(jnp.bfloat16)
  tbl32 = jax.lax.bitcast_convert_type(
      tbl16.reshape(N_NODES, D // 2, 2), jnp.int32)

  self_p, sums, cnts = _sc_aggregate(
      nodes_p, d0, s0, d1, s1, feat_table, tbl32)

  w1t = W1.T
  sig = jnp.asarray(_SIGMA)
  w1t = jnp.concatenate(
      [w1t[:D], w1t[D:2 * D][sig], w1t[2 * D:][sig]], axis=0)
  w2t = W2.T
  b1r = b1.reshape(1, D)
  b2r = b2.reshape(1, D)
  cnts4 = cnts.reshape(2, 2, BP, 1)
  return _tc_mlp(self_p, sums, cnts4, w1t, b1r, w2t, b2r)


# speedup vs baseline: 1.0341x; 1.0341x over previous
import functools

import jax
import jax.numpy as jnp
import numpy as np
from jax import lax
from jax.experimental import pallas as pl
from jax.experimental.pallas import tpu as pltpu
from jax.experimental.pallas import tpu_sc as plsc

N_NODES = 10000
D = 128
B = 10000
BP = 10240
E = 160000
EP = 163840
GW = 64
NG = EP // (32 * GW)
G0 = 96
G1 = 160 - G0
ROWS_PER_TILE = BP // 16
NODE_ROWS = BP // GW

_SIGMA = np.empty((D,), np.int64)
for _k in range(D // 32):
  for _i in range(16):
    _SIGMA[32 * _k + _i] = 32 * _k + 2 * _i
    _SIGMA[32 * _k + 16 + _i] = 32 * _k + 2 * _i + 1


def _sc_aggregate(nodes2d, dst0, src0, dst1, src1, feat_table, tbl32):
  mesh = plsc.VectorSubcoreMesh(
      core_axis_name="c", subcore_axis_name="s", num_cores=2, num_subcores=16)

  @functools.partial(
      pl.kernel,
      out_type=[
          jax.ShapeDtypeStruct((BP, D), jnp.float32),
          jax.ShapeDtypeStruct((2, 2, BP, D), jnp.float32),
          jax.ShapeDtypeStruct((2, 2, BP), jnp.float32),
      ],
      mesh=mesh,
      compiler_params=pltpu.CompilerParams(use_tc_tiling_on_sc=False),
      scratch_types=[
          pltpu.VMEM((max(G0, G1), GW), jnp.int32),
          pltpu.VMEM((max(G0, G1), GW), jnp.int32),
          pltpu.VMEM((NODE_ROWS // 32, GW), jnp.int32),
          pltpu.VMEM((2, GW, D // 2), jnp.int32),
          pltpu.VMEM((2, GW, D), jnp.float32),
          pltpu.VMEM((GW,), jnp.float32),
          pltpu.VMEM((16, D), jnp.float32),
          pltpu.VMEM((ROWS_PER_TILE,), jnp.float32),
          pltpu.VMEM_SHARED((BP, D), jnp.float32),
          pltpu.VMEM_SHARED((BP,), jnp.float32),
          pltpu.SemaphoreType.DMA,
          pltpu.SemaphoreType.DMA((2,)),
          pltpu.SemaphoreType.DMA((2,)),
      ],
  )
  def body(nodes_hbm, d0_hbm, s0_hbm, d1_hbm, s1_hbm, table_hbm, tbl32_hbm,
           self_hbm, sums_hbm, cnts_hbm,
           dstbuf, srcbuf, nodebuf, brows, frows, ones, zrow, zcnt,
           acc, cntacc, sem, gsem, ssem):
    core = lax.axis_index("c")
    sid = lax.axis_index("s")
    wid = sid * 2 + core

    for i in range(GW // 16):
      ones[pl.ds(i * 16, 16)] = jnp.ones((16,), jnp.float32)
    for r in range(16):
      for i in range(8):
        zrow[r, pl.ds(i * 16, 16)] = jnp.zeros((16,), jnp.float32)
    for i in range(ROWS_PER_TILE // 16):
      zcnt[pl.ds(i * 16, 16)] = jnp.zeros((16,), jnp.float32)

    NT = NODE_ROWS // 32
    pltpu.sync_copy(nodes_hbm.at[pl.ds(wid * NT, NT)], nodebuf)
    pltpu.async_copy(table_hbm.at[nodebuf.at[0]], frows.at[0], gsem.at[0])
    pltpu.async_copy(table_hbm.at[nodebuf.at[1]], frows.at[1], gsem.at[1])
    for t in range(NT):
      b = t % 2
      pltpu.make_async_copy(
          table_hbm.at[nodebuf.at[t]], frows.at[b], gsem.at[b]).wait()
      pltpu.sync_copy(
          frows.at[b], self_hbm.at[pl.ds((wid * NT + t) * GW, GW)])
      if t + 2 < NT:
        pltpu.async_copy(
            table_hbm.at[nodebuf.at[t + 2]], frows.at[b], gsem.at[b])

    for rel, (d_hbm, s_hbm) in enumerate(((d0_hbm, s0_hbm), (d1_hbm, s1_hbm))):
      gstart = lax.select(core == 0, sid * G0, 16 * G0 + sid * G1)
      ng = lax.select(core == 0, G0, G1)
      idx_dma_d = pltpu.async_copy(
          d_hbm.at[pl.ds(gstart, max(G0, G1))], dstbuf, sem)
      idx_dma_s = pltpu.async_copy(
          s_hbm.at[pl.ds(gstart, max(G0, G1))], srcbuf, sem)

      def zero_step(j, zc):
        pltpu.sync_copy(zrow, acc.at[pl.ds(sid * ROWS_PER_TILE + j * 16, 16)])
        return zc
      lax.fori_loop(0, ROWS_PER_TILE // 16, zero_step, 0)
      pltpu.sync_copy(zcnt, cntacc.at[pl.ds(sid * ROWS_PER_TILE, ROWS_PER_TILE)])
      idx_dma_d.wait()
      idx_dma_s.wait()
      plsc.subcore_barrier()

      pltpu.async_copy(tbl32_hbm.at[srcbuf.at[0]], brows.at[0], gsem.at[0])
      pltpu.async_copy(tbl32_hbm.at[srcbuf.at[1]], brows.at[1], gsem.at[1])

      def edge_step(j, carry):
        buf = lax.rem(j, 2)
        pltpu.make_async_copy(
            tbl32_hbm.at[srcbuf.at[j]], brows.at[buf], gsem.at[buf]).wait()
        @pl.when(j >= 2)
        def _drain():
          pltpu.make_async_copy(
              frows.at[buf], acc.at[dstbuf.at[j]], ssem.at[buf]).wait()
        for r in range(GW):
          for c in range(D // 32):
            u = brows[buf, r, pl.ds(16 * c, 16)]
            lo = lax.bitcast_convert_type(u << 16, jnp.float32)
            hi = lax.bitcast_convert_type(u & (-65536), jnp.float32)
            frows[buf, r, pl.ds(32 * c, 16)] = lo
            frows[buf, r, pl.ds(32 * c + 16, 16)] = hi
        @pl.when(j + 2 < ng)
        def _prefetch():
          pltpu.async_copy(
              tbl32_hbm.at[srcbuf.at[j + 2]], brows.at[buf], gsem.at[buf])
        pltpu.async_copy(frows.at[buf], acc.at[dstbuf.at[j]], ssem.at[buf],
                         add=True)
        pltpu.sync_copy(ones, cntacc.at[dstbuf.at[j]], add=True)
        return carry
      lax.fori_loop(0, ng, edge_step, 0)
      for b in range(2):
        pltpu.make_async_copy(
            frows.at[b], acc.at[dstbuf.at[b]], ssem.at[b]).wait()
      plsc.subcore_barrier()

      sl = pl.ds(sid * ROWS_PER_TILE, ROWS_PER_TILE)
      pltpu.sync_copy(acc.at[sl], sums_hbm.at[rel, core].at[sl])
      pltpu.sync_copy(cntacc.at[sl], cnts_hbm.at[rel, core].at[sl])

  return body(nodes2d, dst0, src0, dst1, src1, feat_table, tbl32)


def _tc_mlp_body(self_ref, sums_ref, cnts_ref, w1t_ref, b1_ref, w2t_ref,
                 b2_ref, out_ref):
  s0 = sums_ref[0, 0] + sums_ref[0, 1]
  c0 = cnts_ref[0, 0] + cnts_ref[0, 1]
  n0 = s0 / jnp.clip(c0, 1.0, None)
  s1 = sums_ref[1, 0] + sums_ref[1, 1]
  c1 = cnts_ref[1, 0] + cnts_ref[1, 1]
  n1 = s1 / jnp.clip(c1, 1.0, None)
  h = jnp.dot(self_ref[...], w1t_ref[pl.ds(0, D)],
              preferred_element_type=jnp.float32)
  h += jnp.dot(n0, w1t_ref[pl.ds(D, D)], preferred_element_type=jnp.float32)
  h += jnp.dot(n1, w1t_ref[pl.ds(2 * D, D)], preferred_element_type=jnp.float32)
  h = jnp.tanh(h + b1_ref[...])
  out_ref[...] = jnp.dot(h, w2t_ref[...],
                         preferred_element_type=jnp.float32) + b2_ref[...]


def _tc_mlp(self_feats, sums, cnts4, w1t, b1, w2t, b2):
  R = 2000
  return pl.pallas_call(
      _tc_mlp_body,
      grid=(B // R,),
      in_specs=[
          pl.BlockSpec((R, D), lambda i: (i, 0)),
          pl.BlockSpec((2, 2, R, D), lambda i: (0, 0, i, 0)),
          pl.BlockSpec((2, 2, R, 1), lambda i: (0, 0, i, 0)),
          pl.BlockSpec((3 * D, D), lambda i: (0, 0)),
          pl.BlockSpec((1, D), lambda i: (0, 0)),
          pl.BlockSpec((D, D), lambda i: (0, 0)),
          pl.BlockSpec((1, D), lambda i: (0, 0)),
      ],
      out_specs=pl.BlockSpec((R, D), lambda i: (i, 0)),
      out_shape=jax.ShapeDtypeStruct((B, D), jnp.float32),
  )(self_feats, sums, cnts4, w1t, b1, w2t, b2)


def kernel(nodes, edge_index_0, edge_index_1, feat_table, W1, b1, W2, b2):
  i32 = jnp.int32
  nodes_p = jnp.concatenate(
      [nodes.astype(i32), jnp.zeros((BP - B,), i32)]).reshape(NODE_ROWS, GW)

  def prep_edges(ei):
    dst = jnp.concatenate([ei[0].astype(i32), jnp.full((EP - E,), B, i32)])
    src = jnp.concatenate([ei[1].astype(i32), jnp.zeros((EP - E,), i32)])
    return dst.reshape(EP // GW, GW), src.reshape(EP // GW, GW)

  d0, s0 = prep_edges(edge_index_0)
  d1, s1 = prep_edges(edge_index_1)
  tbl16 = feat_table.astype(jnp.bfloat16)
  tbl32 = jax.lax.bitcast_convert_type(
      tbl16.reshape(N_NODES, D // 2, 2), jnp.int32)

  self_p, sums, cnts = _sc_aggregate(
      nodes_p, d0, s0, d1, s1, feat_table, tbl32)

  w1t = W1.T
  sig = jnp.asarray(_SIGMA)
  w1t = jnp.concatenate(
      [w1t[:D], w1t[D:2 * D][sig], w1t[2 * D:][sig]], axis=0)
  w2t = W2.T
  b1r = b1.reshape(1, D)
  b2r = b2.reshape(1, D)
  cnts4 = cnts.reshape(2, 2, BP, 1)
  return _tc_mlp(self_p, sums, cnts4, w1t, b1r, w2t, b2r)
